# Initial kernel scaffold; baseline (speedup 1.0000x reference)
import jax
import jax.numpy as jnp
from jax.experimental import pallas as pl


def kernel(x, edge_index, edge_attr, Wl, bl, Wr, ln_w, ln_b, Wres, bres, Wfc, bfc):
    raise NotImplementedError("write your pallas kernel here")


# speedup vs baseline: 7.2377x; 7.2377x over previous
import functools

import jax
import jax.numpy as jnp
from jax import lax
from jax.experimental import pallas as pl
from jax.experimental.pallas import tpu as pltpu
from jax.experimental.pallas import tpu_sc as plsc

F32 = jnp.float32

NC = 2
NS = 16
NW = NC * NS
CH = 80
GR = 80
LANES = 16


@functools.cache
def _make_sc_agg(n_nodes, feat, n_edges, compute_deg):
    epw = n_edges // NW
    assert epw * NW == n_edges and epw % CH == 0
    sup = min(epw, 2000)
    assert epw % sup == 0 and sup % CH == 0
    n_sup = epw // sup
    chunks_per_sup = sup // CH
    assert n_nodes % GR == 0
    n_groups = n_nodes // GR
    gpt = pl.cdiv(n_groups, NS)
    fpl = feat // LANES

    mesh = plsc.VectorSubcoreMesh(core_axis_name="c", subcore_axis_name="s")

    out_type = [jax.ShapeDtypeStruct((NC, n_nodes, feat), F32)]
    if compute_deg:
        out_type.append(jax.ShapeDtypeStruct((NW * n_nodes,), F32))

    scratch = [
        pltpu.VMEM_SHARED((n_nodes, feat), F32),
        pltpu.VMEM((sup,), jnp.int32),
        pltpu.VMEM((sup,), jnp.int32),
        pltpu.VMEM((CH,), jnp.int32),
        pltpu.VMEM((CH,), jnp.int32),
        pltpu.VMEM((CH, feat), F32),
        pltpu.VMEM((GR, feat), F32),
        pltpu.SemaphoreType.DMA,
    ]
    if compute_deg:
        scratch.append(pltpu.VMEM((n_nodes,), F32))

    def body(h_hbm, src_hbm, dst_hbm, *rest):
        if compute_deg:
            acc_out, deg_out = rest[0], rest[1]
            acc, srcs, dsts, src_c, dst_c, rows, stage, sem, degl = rest[2:]
        else:
            acc_out = rest[0]
            acc, srcs, dsts, src_c, dst_c, rows, stage, sem = rest[1:]

        cid = lax.axis_index("c")
        sid = lax.axis_index("s")
        wid = sid * NC + cid

        zero16 = jnp.zeros((LANES,), F32)
        ones16 = jnp.ones((LANES,), F32)

        def zb(i, _):
            stage[i // fpl, pl.ds((i % fpl) * LANES, LANES)] = zero16
            return 0
        lax.fori_loop(0, GR * fpl, zb, 0)

        for k in range(gpt):
            g = sid + k * NS

            @pl.when(g < n_groups)
            def _():
                pltpu.sync_copy(stage, acc.at[pl.ds(g * GR, GR)])

        if compute_deg:
            def zdeg(i, _):
                degl[pl.ds(i * LANES, LANES)] = zero16
                return 0
            lax.fori_loop(0, n_nodes // LANES, zdeg, 0)

        plsc.subcore_barrier()

        def superchunk(si, _):
            sbase = wid * epw + si * sup
            pltpu.sync_copy(src_hbm.at[pl.ds(sbase, sup)], srcs)
            pltpu.sync_copy(dst_hbm.at[pl.ds(sbase, sup)], dsts)

            def chunk(ci, _):
                off = ci * CH
                for j in range(CH // LANES):
                    src_c[pl.ds(j * LANES, LANES)] = srcs[
                        pl.ds(off + j * LANES, LANES)]
                    dst_c[pl.ds(j * LANES, LANES)] = dsts[
                        pl.ds(off + j * LANES, LANES)]
                pltpu.async_copy(h_hbm.at[src_c], rows, sem).wait()
                pltpu.sync_copy(rows, acc.at[dst_c], add=True)
                if compute_deg:
                    for j in range(CH // LANES):
                        idx = dst_c[pl.ds(j * LANES, LANES)]
                        plsc.addupdate_scatter(degl, [idx], ones16)
                return 0
            lax.fori_loop(0, chunks_per_sup, chunk, 0)
            return 0
        lax.fori_loop(0, n_sup, superchunk, 0)

        plsc.subcore_barrier()

        for k in range(gpt):
            g = sid + k * NS

            @pl.when(g < n_groups)
            def _():
                pltpu.sync_copy(acc.at[pl.ds(g * GR, GR)], stage)
                pltpu.sync_copy(stage, acc_out.at[cid, pl.ds(g * GR, GR)])

        if compute_deg:
            pltpu.sync_copy(degl, deg_out.at[pl.ds(wid * n_nodes, n_nodes)])

    return pl.kernel(
        body,
        out_type=tuple(out_type) if compute_deg else out_type[0],
        mesh=mesh,
        scratch_types=scratch,
        compiler_params=pltpu.CompilerParams(needs_layout_passes=False),
    )


@functools.cache
def _make_tc_layer(n_nodes, feat, out_feat, br, has_res_proj, has_fc):
    grid = (pl.cdiv(n_nodes, br),)

    def body(h_ref, acc_ref, deg_ref, wl_ref, bl_ref, wr_ref, lnw_ref,
             lnb_ref, *rest):
        o_ref = rest[-1]
        h = h_ref[...]
        agg = acc_ref[0] + acc_ref[1]
        deg = jnp.sum(deg_ref[...], axis=-1, keepdims=True)
        mean = agg * (1.0 / jnp.maximum(deg, 1.0))
        z = jnp.dot(mean, wl_ref[...], preferred_element_type=F32)
        z = z + bl_ref[...]
        z = z + jnp.dot(h, wr_ref[...], preferred_element_type=F32)
        mu = jnp.mean(z, axis=-1, keepdims=True)
        zc = z - mu
        var = jnp.mean(zc * zc, axis=-1, keepdims=True)
        z = zc * lax.rsqrt(var + 1e-5) * lnw_ref[...] + lnb_ref[...]
        z = jnp.maximum(z, 0.0)
        if has_res_proj:
            res = jnp.dot(h, rest[0][...], preferred_element_type=F32)
            res = res + rest[1][...]
        else:
            res = h
        hn = z + res
        if has_fc:
            k = 2 if has_res_proj else 0
            hn = jnp.dot(hn, rest[k][...], preferred_element_type=F32)
            hn = hn + rest[k + 1][...]
        o_ref[...] = hn

    w_spec = pl.BlockSpec((feat, feat), lambda i: (0, 0))
    b_spec = pl.BlockSpec((1, feat), lambda i: (0, 0))
    in_specs = [
        pl.BlockSpec((br, feat), lambda i: (i, 0)),
        pl.BlockSpec((NC, br, feat), lambda i: (0, i, 0)),
        pl.BlockSpec((br, NW), lambda i: (i, 0)),
        w_spec, b_spec, w_spec, b_spec, b_spec,
    ]
    if has_res_proj:
        in_specs += [w_spec, b_spec]
    if has_fc:
        in_specs += [pl.BlockSpec((feat, out_feat), lambda i: (0, 0)),
                     pl.BlockSpec((1, out_feat), lambda i: (0, 0))]

    return pl.pallas_call(
        body,
        grid=grid,
        in_specs=in_specs,
        out_specs=pl.BlockSpec((br, out_feat), lambda i: (i, 0)),
        out_shape=jax.ShapeDtypeStruct((n_nodes, out_feat), F32),
    )


def kernel(x, edge_index, edge_attr, Wl, bl, Wr, ln_w, ln_b, Wres, bres,
           Wfc, bfc):
    n_nodes, feat = x.shape
    n_edges = edge_index.shape[1]
    out_feat = Wfc.shape[1]
    br = 512

    src = edge_index[0]
    dst = edge_index[1]

    sc_first = _make_sc_agg(n_nodes, feat, n_edges, True)
    sc_rest = _make_sc_agg(n_nodes, feat, n_edges, False)

    accp, degp = sc_first(x, src, dst)
    degt = degp.reshape(NW, n_nodes).T

    tc0 = _make_tc_layer(n_nodes, feat, feat, br, True, False)
    h = tc0(x, accp, degt, Wl[0], bl[0][None, :], Wr[0], ln_w[0][None, :],
            ln_b[0][None, :], Wres, bres[None, :])

    tc_mid = _make_tc_layer(n_nodes, feat, feat, br, False, False)
    accp = sc_rest(h, src, dst)
    h = tc_mid(h, accp, degt, Wl[1], bl[1][None, :], Wr[1],
               ln_w[1][None, :], ln_b[1][None, :])

    tc_last = _make_tc_layer(n_nodes, feat, out_feat, br, False, True)
    accp = sc_rest(h, src, dst)
    out = tc_last(h, accp, degt, Wl[2], bl[2][None, :], Wr[2],
                  ln_w[2][None, :], ln_b[2][None, :], Wfc, bfc[None, :])
    return out
